# baseline (speedup 1.0000x reference)
import functools

import jax
import jax.numpy as jnp
import numpy as np
from jax import lax
from jax.experimental import pallas as pl
from jax.experimental.pallas import tpu as pltpu
from jax.experimental.pallas import tpu_sc as plsc

_N = 10000
_NC, _NS = 2, 16
_B = 120
_NPAD = 10112
_RPS = _NPAD // _NS
_AW = 144


def _mm_cat(x, w, expmat):
    n, k = x.shape
    f = w.shape[1]
    fe = expmat.shape[1]
    br = 1000

    def body(xr, wr, er, outr):
        xl = jnp.dot(xr[...], wr[...], preferred_element_type=jnp.float32)
        av = jnp.dot(xl, er[...], preferred_element_type=jnp.float32)
        outr[...] = jnp.concatenate([xl, av], axis=1)

    return pl.pallas_call(
        body,
        grid=(n // br,),
        in_specs=[
            pl.BlockSpec((br, k), lambda i: (i, 0)),
            pl.BlockSpec((k, f), lambda i: (0, 0)),
            pl.BlockSpec((f, fe), lambda i: (0, 0)),
        ],
        out_specs=pl.BlockSpec((br, f + fe), lambda i: (i, 0)),
        out_shape=jax.ShapeDtypeStruct((n, f + fe), jnp.float32),
    )(x, w, expmat)


def _epilogue(accs, exps, b, res):
    npass = len(accs)
    f = 128 * npass
    br = 1000
    nres = 1 if res is not None else 0

    def body(*refs):
        arefs = refs[:npass]
        erefs = refs[npass:2 * npass]
        bref = refs[2 * npass]
        rref = refs[2 * npass + 1] if nres else None
        outr = refs[-1]
        cols = []
        for p in range(npass):
            s = arefs[p][0] + arefs[p][1]
            den = jnp.dot(s[:, 128:136], erefs[p][...],
                          preferred_element_type=jnp.float32)
            cols.append(s[:, :128] / (den + 1e-16))
        x = cols[0] if npass == 1 else jnp.concatenate(cols, axis=1)
        x = x + bref[...]
        x = jnp.where(x > 0, x, jnp.exp(jnp.minimum(x, 0.0)) - 1.0)
        if nres:
            x = x + rref[...]
        outr[...] = x

    in_specs = (
        [pl.BlockSpec((2, br, _AW), lambda i: (0, i, 0)) for _ in range(npass)]
        + [pl.BlockSpec((8, 128), lambda i: (0, 0)) for _ in range(npass)]
        + [pl.BlockSpec((1, f), lambda i: (0, 0))]
        + ([pl.BlockSpec((br, f), lambda i: (i, 0))] if nres else [])
    )
    args = list(accs) + list(exps) + [b.reshape(1, f)] + ([res] if nres else [])
    return pl.pallas_call(
        body,
        grid=(_N // br,),
        in_specs=in_specs,
        out_specs=pl.BlockSpec((br, f), lambda i: (i, 0)),
        out_shape=jax.ShapeDtypeStruct((_N, f), jnp.float32),
    )(*args)


def _bn_stats(h, g):
    def body(hr, gr, outr):
        hv = hr[...]
        gv = gr[...]
        mu_h = jnp.mean(hv, axis=0, keepdims=True)
        mu_g = jnp.mean(gv, axis=0, keepdims=True)
        var_h = jnp.mean((hv - mu_h) ** 2, axis=0, keepdims=True)
        var_g = jnp.mean((gv - mu_g) ** 2, axis=0, keepdims=True)
        is_h = lax.rsqrt(var_h + 1e-5)
        is_g = lax.rsqrt(var_g + 1e-5)
        z = jnp.zeros_like(mu_h)
        outr[...] = jnp.concatenate(
            [mu_h, is_h, mu_g, is_g, z, z, z, z], axis=0)

    return pl.pallas_call(
        body,
        out_shape=jax.ShapeDtypeStruct((8, 128), jnp.float32),
    )(h, g)


def _head(h, g, stats, bnc, bns, caw1, cab1, caw2p, cab2, saw1, sab1, saw2p,
          sab2, fc1wp, fc1bp, fc2wp, fc2bp, fc3wp, fc3bp):
    br = 1000

    def body(hr, gr, str_, bncr, bnsr, cw1, cb1, cw2, cb2, sw1, sb1, sw2, sb2,
             f1w, f1b, f2w, f2b, f3w, f3b, outr):
        st = str_[...]
        hv = (hr[...] - st[0:1]) * st[1:2] * bncr[0:1] + bncr[1:2]
        gv = (gr[...] - st[2:3]) * st[3:4] * bnsr[0:1] + bnsr[1:2]
        cs = jnp.maximum(
            jnp.dot(hv, cw1[...], preferred_element_type=jnp.float32)
            + cb1[...], 0.0)
        cs = jnp.dot(cs, cw2[...], preferred_element_type=jnp.float32) + cb2[...]
        cs = jax.nn.sigmoid(cs[:, 0:1])
        ss = jnp.maximum(
            jnp.dot(gv, sw1[...], preferred_element_type=jnp.float32)
            + sb1[...], 0.0)
        ss = jnp.dot(ss, sw2[...], preferred_element_type=jnp.float32) + sb2[...]
        ss = jax.nn.sigmoid(ss[:, 0:1])
        mx = jnp.maximum(cs, ss)
        e0 = jnp.exp(cs - mx)
        e1 = jnp.exp(ss - mx)
        aw0 = e0 / (e0 + e1)
        aw1 = e1 / (e0 + e1)
        x = jnp.concatenate([hv * aw0, gv * aw1], axis=1)
        x = jnp.dot(x, f1w[...], preferred_element_type=jnp.float32) + f1b[...]
        x = jnp.where(x > 0, x, jnp.exp(jnp.minimum(x, 0.0)) - 1.0)
        x = jnp.dot(x, f2w[...], preferred_element_type=jnp.float32) + f2b[...]
        x = jnp.where(x > 0, x, jnp.exp(jnp.minimum(x, 0.0)) - 1.0)
        x = jnp.dot(x, f3w[...], preferred_element_type=jnp.float32) + f3b[...]
        col = lax.broadcasted_iota(jnp.int32, x.shape, 1)
        xm = jnp.where(col < 40, x, -1e30)
        m = jnp.max(xm, axis=1, keepdims=True)
        lse = jnp.log(jnp.sum(jnp.exp(xm - m), axis=1, keepdims=True))
        outr[...] = xm - m - lse

    full = lambda a: pl.BlockSpec(a.shape, lambda i: tuple(0 for _ in a.shape))
    weights = [caw1, cab1, caw2p, cab2, saw1, sab1, saw2p, sab2,
               fc1wp, fc1bp, fc2wp, fc2bp, fc3wp, fc3bp]
    return pl.pallas_call(
        body,
        grid=(_N // br,),
        in_specs=(
            [pl.BlockSpec((br, 128), lambda i: (i, 0)),
             pl.BlockSpec((br, 128), lambda i: (i, 0)),
             full(stats), full(bnc), full(bns)]
            + [full(w) for w in weights]),
        out_specs=pl.BlockSpec((br, 128), lambda i: (i, 0)),
        out_shape=jax.ShapeDtypeStruct((_N, 128), jnp.float32),
    )(h, g, stats, bnc, bns, *weights)


def _edge_pass(xcat, adst_t, sd_i, zrows, steps, head_of_vec):
    per_tile = steps * _B
    assert steps % 2 == 0
    mesh = plsc.VectorSubcoreMesh(core_axis_name="c", subcore_axis_name="s",
                                  num_cores=_NC, num_subcores=_NS)

    @functools.partial(
        pl.kernel,
        mesh=mesh,
        compiler_params=pltpu.CompilerParams(use_tc_tiling_on_sc=False),
        out_type=jax.ShapeDtypeStruct((_NC, _NPAD, _AW), jnp.float32),
        scratch_types=[
            pltpu.VMEM_SHARED((_NPAD, _AW), jnp.float32),
            pltpu.VMEM((2, 2, _B), jnp.int32),
            pltpu.VMEM((2, _B, _AW), jnp.float32),
            pltpu.VMEM((2, _B, 16), jnp.float32),
            pltpu.SemaphoreType.DMA,
            pltpu.SemaphoreType.DMA,
            pltpu.SemaphoreType.DMA,
            pltpu.SemaphoreType.DMA,
        ],
    )
    def kfn(xcat_h, adst_h, sd_h, zr_h, out_h,
            acc, sdv, gbuf, dbuf, s1a, s2a, s1b, s2b):
        cid = lax.axis_index("c")
        sid = lax.axis_index("s")
        pltpu.sync_copy(zr_h, acc.at[pl.ds(sid * _RPS, _RPS)])
        plsc.subcore_barrier()
        tile = cid * _NS + sid
        sems = ((s1a, s2a), (s1b, s2b))

        def fetch(i, b):
            g = tile * steps + i
            pltpu.sync_copy(sd_h.at[pl.ds(2 * g, 2)], sdv.at[b])
            pltpu.async_copy(xcat_h.at[sdv.at[b, 0]], gbuf.at[b], sems[b][0])
            pltpu.async_copy(adst_h.at[sdv.at[b, 1]], dbuf.at[b], sems[b][1])

        def consume(i, b):
            pltpu.make_async_copy(xcat_h.at[sdv.at[b, 0]], gbuf.at[b],
                                  sems[b][0]).wait()
            pltpu.make_async_copy(adst_h.at[sdv.at[b, 1]], dbuf.at[b],
                                  sems[b][1]).wait()

            def edge(e, c2):
                asv = gbuf[b, e, pl.ds(128, 16)]
                adv = dbuf[b, e, pl.ds(0, 16)]
                al = asv + adv
                al = jnp.maximum(al, al * 0.2)
                w = jnp.exp(al)
                for j in range(8):
                    wj = w[head_of_vec[j]]
                    gbuf[b, e, pl.ds(j * 16, 16)] = (
                        gbuf[b, e, pl.ds(j * 16, 16)] * wj)
                gbuf[b, e, pl.ds(128, 16)] = w
                return c2

            lax.fori_loop(0, _B, edge, 0, unroll=8)
            pltpu.sync_copy(gbuf.at[b], acc.at[sdv.at[b, 1]], add=True)

        fetch(0, 0)

        def blk2(i2, carry):
            i = i2 * 2
            for b in range(2):

                @pl.when(i + b + 1 < steps)
                def _():
                    fetch(i + b + 1, 1 - b)

                consume(i + b, b)
            return carry

        lax.fori_loop(0, steps // 2, blk2, 0)
        plsc.subcore_barrier()
        pltpu.sync_copy(acc.at[pl.ds(sid * _RPS, _RPS)],
                        out_h.at[cid, pl.ds(sid * _RPS, _RPS)])

    return kfn(xcat, adst_t, sd_i, zrows)


def _place(a):
    h, c = a.shape
    out = jnp.zeros((h * c, h), jnp.float32)
    for i in range(h):
        out = out.at[i * c:(i + 1) * c, i].set(a[i])
    return out


def _head_exp(head_of_col):
    m = np.zeros((8, 128), np.float32)
    for c, h in enumerate(head_of_col):
        m[h, c] = 1.0
    return jnp.asarray(m)


def _gat_conv(x, sd_i, zrows, steps, n_valid, W, a_s, a_d, b, res):
    n = x.shape[0]
    f = W.shape[1]
    c = a_s.shape[1]
    expmat = jnp.concatenate([_place(a_s), _place(a_d)], axis=1)
    xlav = _mm_cat(x, W, expmat)
    prow = jnp.zeros((8, f + 16), jnp.float32).at[:, f:f + 8].set(-1e30)
    xlav = jnp.concatenate([xlav, prow], axis=0)
    zpad = jnp.zeros((n + 8, 8), jnp.float32)
    adst_t = jnp.concatenate([xlav[:, f + 8:f + 16], zpad], axis=1)
    npass = f // 128
    accs, exps = [], []
    for p in range(npass):
        xcat = jnp.concatenate(
            [xlav[:, p * 128:(p + 1) * 128], xlav[:, f:f + 8], zpad], axis=1)
        head_of_vec = tuple((p * 128 + 16 * j) // c for j in range(8))
        acc = _edge_pass(xcat, adst_t, sd_i, zrows, steps, head_of_vec)
        accs.append(acc[:, :n, :])
        exps.append(_head_exp([(p * 128 + cc) // c for cc in range(128)]))
    return _epilogue(accs, exps, b, res)


def kernel(content_x, social_x, content_edge_index, social_edge_index,
           Wc1, ac1s, ac1d, bc1, Wc2, ac2s, ac2d, bc2,
           Ws1, as1s, as1d, bs1, Ws2, as2s, as2d, bs2,
           ca_w1, ca_b1, ca_w2, ca_b2, sa_w1, sa_b1, sa_w2, sa_b2,
           bnc_g, bnc_b, bns_g, bns_b,
           fc1_w, fc1_b, fc2_w, fc2_b, fc3_w, fc3_b):
    n = content_x.shape[0]
    e = content_edge_index.shape[1]
    n_valid = e + n
    steps = -(-n_valid // (_NC * _NS * _B))
    steps = steps + (steps % 2)
    ep = _NC * _NS * steps * _B
    loop = jnp.arange(n, dtype=content_edge_index.dtype)
    pad = jnp.full((ep - n_valid,), n, content_edge_index.dtype)

    def edges(ei):
        s = jnp.concatenate([ei[0], loop, pad])
        d = jnp.concatenate([ei[1], loop, pad])
        nb = ep // _B
        return jnp.stack([s.reshape(nb, _B), d.reshape(nb, _B)],
                         axis=1).reshape(2 * nb, _B)

    csd = edges(content_edge_index)
    ssd = edges(social_edge_index)
    zrows = jnp.zeros((_RPS, _AW), jnp.float32)

    h = _gat_conv(content_x, csd, zrows, steps, n_valid,
                  Wc1, ac1s, ac1d, bc1, None)
    h = _gat_conv(h, csd, zrows, steps, n_valid,
                  Wc2, ac2s, ac2d, bc2, None)
    g = _gat_conv(social_x, ssd, zrows, steps, n_valid,
                  Ws1, as1s, as1d, bs1, social_x)
    g = _gat_conv(g, ssd, zrows, steps, n_valid,
                  Ws2, as2s, as2d, bs2, None)

    stats = _bn_stats(h, g)
    bnc = jnp.stack([bnc_g, bnc_b])
    bns = jnp.stack([bns_g, bns_b])

    pad_w = lambda w, rows, cols: jnp.zeros((rows, cols), jnp.float32).at[
        :w.shape[0], :w.shape[1]].set(w)
    pad_b = lambda b, cols: jnp.zeros((1, cols), jnp.float32).at[
        0, :b.shape[0]].set(b)

    out = _head(
        h, g, stats, bnc, bns,
        ca_w1, ca_b1.reshape(1, -1), pad_w(ca_w2, 64, 128),
        pad_b(ca_b2, 128), sa_w1, sa_b1.reshape(1, -1),
        pad_w(sa_w2, 64, 128), pad_b(sa_b2, 128),
        pad_w(fc1_w, 256, 128), pad_b(fc1_b, 128),
        pad_w(fc2_w, 128, 128), pad_b(fc2_b, 128),
        pad_w(fc3_w, 128, 128), pad_b(fc3_b, 128))
    return out[:, :40]


# speedup vs baseline: 1.0957x; 1.0957x over previous
import functools

import jax
import jax.numpy as jnp
import numpy as np
from jax import lax
from jax.experimental import pallas as pl
from jax.experimental.pallas import tpu as pltpu
from jax.experimental.pallas import tpu_sc as plsc

_N = 10000
_NC, _NS = 2, 16
_B = 120
_NPAD = 10112
_RPS = _NPAD // _NS
_AW = 144


def _mm_cat(x, w, expmat):
    n, k = x.shape
    f = w.shape[1]
    fe = expmat.shape[1]
    br = 1000

    def body(xr, wr, er, outr):
        xl = jnp.dot(xr[...], wr[...], preferred_element_type=jnp.float32)
        av = jnp.dot(xl, er[...], preferred_element_type=jnp.float32)
        outr[...] = jnp.concatenate([xl, av], axis=1)

    return pl.pallas_call(
        body,
        grid=(n // br,),
        in_specs=[
            pl.BlockSpec((br, k), lambda i: (i, 0)),
            pl.BlockSpec((k, f), lambda i: (0, 0)),
            pl.BlockSpec((f, fe), lambda i: (0, 0)),
        ],
        out_specs=pl.BlockSpec((br, f + fe), lambda i: (i, 0)),
        out_shape=jax.ShapeDtypeStruct((n, f + fe), jnp.float32),
    )(x, w, expmat)


def _epilogue(accs, exps, b, res):
    npass = len(accs)
    f = 128 * npass
    br = 1000
    nres = 1 if res is not None else 0

    def body(*refs):
        arefs = refs[:npass]
        erefs = refs[npass:2 * npass]
        bref = refs[2 * npass]
        rref = refs[2 * npass + 1] if nres else None
        outr = refs[-1]
        cols = []
        for p in range(npass):
            s = arefs[p][0] + arefs[p][1]
            den = jnp.dot(s[:, 128:136], erefs[p][...],
                          preferred_element_type=jnp.float32)
            cols.append(s[:, :128] / (den + 1e-16))
        x = cols[0] if npass == 1 else jnp.concatenate(cols, axis=1)
        x = x + bref[...]
        x = jnp.where(x > 0, x, jnp.exp(jnp.minimum(x, 0.0)) - 1.0)
        if nres:
            x = x + rref[...]
        outr[...] = x

    in_specs = (
        [pl.BlockSpec((2, br, _AW), lambda i: (0, i, 0)) for _ in range(npass)]
        + [pl.BlockSpec((8, 128), lambda i: (0, 0)) for _ in range(npass)]
        + [pl.BlockSpec((1, f), lambda i: (0, 0))]
        + ([pl.BlockSpec((br, f), lambda i: (i, 0))] if nres else [])
    )
    args = list(accs) + list(exps) + [b.reshape(1, f)] + ([res] if nres else [])
    return pl.pallas_call(
        body,
        grid=(_N // br,),
        in_specs=in_specs,
        out_specs=pl.BlockSpec((br, f), lambda i: (i, 0)),
        out_shape=jax.ShapeDtypeStruct((_N, f), jnp.float32),
    )(*args)


def _bn_stats(h, g):
    def body(hr, gr, outr):
        hv = hr[...]
        gv = gr[...]
        mu_h = jnp.mean(hv, axis=0, keepdims=True)
        mu_g = jnp.mean(gv, axis=0, keepdims=True)
        var_h = jnp.mean((hv - mu_h) ** 2, axis=0, keepdims=True)
        var_g = jnp.mean((gv - mu_g) ** 2, axis=0, keepdims=True)
        is_h = lax.rsqrt(var_h + 1e-5)
        is_g = lax.rsqrt(var_g + 1e-5)
        z = jnp.zeros_like(mu_h)
        outr[...] = jnp.concatenate(
            [mu_h, is_h, mu_g, is_g, z, z, z, z], axis=0)

    return pl.pallas_call(
        body,
        out_shape=jax.ShapeDtypeStruct((8, 128), jnp.float32),
    )(h, g)


def _head(h, g, stats, bnc, bns, caw1, cab1, caw2p, cab2, saw1, sab1, saw2p,
          sab2, fc1wp, fc1bp, fc2wp, fc2bp, fc3wp, fc3bp):
    br = 1000

    def body(hr, gr, str_, bncr, bnsr, cw1, cb1, cw2, cb2, sw1, sb1, sw2, sb2,
             f1w, f1b, f2w, f2b, f3w, f3b, outr):
        st = str_[...]
        hv = (hr[...] - st[0:1]) * st[1:2] * bncr[0:1] + bncr[1:2]
        gv = (gr[...] - st[2:3]) * st[3:4] * bnsr[0:1] + bnsr[1:2]
        cs = jnp.maximum(
            jnp.dot(hv, cw1[...], preferred_element_type=jnp.float32)
            + cb1[...], 0.0)
        cs = jnp.dot(cs, cw2[...], preferred_element_type=jnp.float32) + cb2[...]
        cs = jax.nn.sigmoid(cs[:, 0:1])
        ss = jnp.maximum(
            jnp.dot(gv, sw1[...], preferred_element_type=jnp.float32)
            + sb1[...], 0.0)
        ss = jnp.dot(ss, sw2[...], preferred_element_type=jnp.float32) + sb2[...]
        ss = jax.nn.sigmoid(ss[:, 0:1])
        mx = jnp.maximum(cs, ss)
        e0 = jnp.exp(cs - mx)
        e1 = jnp.exp(ss - mx)
        aw0 = e0 / (e0 + e1)
        aw1 = e1 / (e0 + e1)
        x = jnp.concatenate([hv * aw0, gv * aw1], axis=1)
        x = jnp.dot(x, f1w[...], preferred_element_type=jnp.float32) + f1b[...]
        x = jnp.where(x > 0, x, jnp.exp(jnp.minimum(x, 0.0)) - 1.0)
        x = jnp.dot(x, f2w[...], preferred_element_type=jnp.float32) + f2b[...]
        x = jnp.where(x > 0, x, jnp.exp(jnp.minimum(x, 0.0)) - 1.0)
        x = jnp.dot(x, f3w[...], preferred_element_type=jnp.float32) + f3b[...]
        col = lax.broadcasted_iota(jnp.int32, x.shape, 1)
        xm = jnp.where(col < 40, x, -1e30)
        m = jnp.max(xm, axis=1, keepdims=True)
        lse = jnp.log(jnp.sum(jnp.exp(xm - m), axis=1, keepdims=True))
        outr[...] = xm - m - lse

    full = lambda a: pl.BlockSpec(a.shape, lambda i: tuple(0 for _ in a.shape))
    weights = [caw1, cab1, caw2p, cab2, saw1, sab1, saw2p, sab2,
               fc1wp, fc1bp, fc2wp, fc2bp, fc3wp, fc3bp]
    return pl.pallas_call(
        body,
        grid=(_N // br,),
        in_specs=(
            [pl.BlockSpec((br, 128), lambda i: (i, 0)),
             pl.BlockSpec((br, 128), lambda i: (i, 0)),
             full(stats), full(bnc), full(bns)]
            + [full(w) for w in weights]),
        out_specs=pl.BlockSpec((br, 128), lambda i: (i, 0)),
        out_shape=jax.ShapeDtypeStruct((_N, 128), jnp.float32),
    )(h, g, stats, bnc, bns, *weights)


def _edge_pass(xcat, adst_t, sd_i, zrows, steps, head_of_vec):
    per_tile = steps * _B
    assert steps % 2 == 0
    mesh = plsc.VectorSubcoreMesh(core_axis_name="c", subcore_axis_name="s",
                                  num_cores=_NC, num_subcores=_NS)

    @functools.partial(
        pl.kernel,
        mesh=mesh,
        compiler_params=pltpu.CompilerParams(use_tc_tiling_on_sc=False),
        out_type=jax.ShapeDtypeStruct((_NC, _NPAD, _AW), jnp.float32),
        scratch_types=[
            pltpu.VMEM_SHARED((_NPAD, _AW), jnp.float32),
            pltpu.VMEM((2, 2, _B), jnp.int32),
            pltpu.VMEM((2, _B, _AW), jnp.float32),
            pltpu.VMEM((2, _B, 16), jnp.float32),
            pltpu.SemaphoreType.DMA,
            pltpu.SemaphoreType.DMA,
            pltpu.SemaphoreType.DMA,
            pltpu.SemaphoreType.DMA,
        ],
    )
    def kfn(xcat_h, adst_h, sd_h, zr_h, out_h,
            acc, sdv, gbuf, dbuf, s1a, s2a, s1b, s2b):
        cid = lax.axis_index("c")
        sid = lax.axis_index("s")
        pltpu.sync_copy(zr_h, acc.at[pl.ds(sid * _RPS, _RPS)])
        plsc.subcore_barrier()
        tile = cid * _NS + sid
        sems = ((s1a, s2a), (s1b, s2b))

        def fetch(i, b):
            g = tile * steps + i
            pltpu.sync_copy(sd_h.at[pl.ds(2 * g, 2)], sdv.at[b])
            pltpu.async_copy(xcat_h.at[sdv.at[b, 0]], gbuf.at[b], sems[b][0])
            pltpu.async_copy(adst_h.at[sdv.at[b, 1]], dbuf.at[b], sems[b][1])

        def consume(i, b):
            pltpu.make_async_copy(xcat_h.at[sdv.at[b, 0]], gbuf.at[b],
                                  sems[b][0]).wait()
            pltpu.make_async_copy(adst_h.at[sdv.at[b, 1]], dbuf.at[b],
                                  sems[b][1]).wait()

            def edge(e, c2):
                asv = gbuf[b, e, pl.ds(128, 16)]
                adv = dbuf[b, e, pl.ds(0, 16)]
                al = asv + adv
                al = jnp.maximum(al, al * 0.2)
                w = jnp.exp(al)
                for j in range(8):
                    wj = w[head_of_vec[j]]
                    gbuf[b, e, pl.ds(j * 16, 16)] = (
                        gbuf[b, e, pl.ds(j * 16, 16)] * wj)
                gbuf[b, e, pl.ds(128, 16)] = w
                return c2

            lax.fori_loop(0, _B, edge, 0, unroll=4)
            pltpu.sync_copy(gbuf.at[b], acc.at[sdv.at[b, 1]], add=True)

        fetch(0, 0)

        def blk2(i2, carry):
            i = i2 * 2
            for b in range(2):

                @pl.when(i + b + 1 < steps)
                def _():
                    fetch(i + b + 1, 1 - b)

                consume(i + b, b)
            return carry

        lax.fori_loop(0, steps // 2, blk2, 0)
        plsc.subcore_barrier()
        pltpu.sync_copy(acc.at[pl.ds(sid * _RPS, _RPS)],
                        out_h.at[cid, pl.ds(sid * _RPS, _RPS)])

    return kfn(xcat, adst_t, sd_i, zrows)


def _place(a):
    h, c = a.shape
    out = jnp.zeros((h * c, h), jnp.float32)
    for i in range(h):
        out = out.at[i * c:(i + 1) * c, i].set(a[i])
    return out


def _head_exp(head_of_col):
    m = np.zeros((8, 128), np.float32)
    for c, h in enumerate(head_of_col):
        m[h, c] = 1.0
    return jnp.asarray(m)


def _gat_conv(x, sd_i, zrows, steps, n_valid, W, a_s, a_d, b, res):
    n = x.shape[0]
    f = W.shape[1]
    c = a_s.shape[1]
    expmat = jnp.concatenate([_place(a_s), _place(a_d)], axis=1)
    xlav = _mm_cat(x, W, expmat)
    prow = jnp.zeros((8, f + 16), jnp.float32).at[:, f:f + 8].set(-1e30)
    xlav = jnp.concatenate([xlav, prow], axis=0)
    zpad = jnp.zeros((n + 8, 8), jnp.float32)
    adst_t = jnp.concatenate([xlav[:, f + 8:f + 16], zpad], axis=1)
    npass = f // 128
    accs, exps = [], []
    for p in range(npass):
        xcat = jnp.concatenate(
            [xlav[:, p * 128:(p + 1) * 128], xlav[:, f:f + 8], zpad], axis=1)
        head_of_vec = tuple((p * 128 + 16 * j) // c for j in range(8))
        acc = _edge_pass(xcat, adst_t, sd_i, zrows, steps, head_of_vec)
        accs.append(acc[:, :n, :])
        exps.append(_head_exp([(p * 128 + cc) // c for cc in range(128)]))
    return _epilogue(accs, exps, b, res)


def kernel(content_x, social_x, content_edge_index, social_edge_index,
           Wc1, ac1s, ac1d, bc1, Wc2, ac2s, ac2d, bc2,
           Ws1, as1s, as1d, bs1, Ws2, as2s, as2d, bs2,
           ca_w1, ca_b1, ca_w2, ca_b2, sa_w1, sa_b1, sa_w2, sa_b2,
           bnc_g, bnc_b, bns_g, bns_b,
           fc1_w, fc1_b, fc2_w, fc2_b, fc3_w, fc3_b):
    n = content_x.shape[0]
    e = content_edge_index.shape[1]
    n_valid = e + n
    steps = -(-n_valid // (_NC * _NS * _B))
    steps = steps + (steps % 2)
    ep = _NC * _NS * steps * _B
    loop = jnp.arange(n, dtype=content_edge_index.dtype)
    pad = jnp.full((ep - n_valid,), n, content_edge_index.dtype)

    def edges(ei):
        s = jnp.concatenate([ei[0], loop, pad])
        d = jnp.concatenate([ei[1], loop, pad])
        nb = ep // _B
        return jnp.stack([s.reshape(nb, _B), d.reshape(nb, _B)],
                         axis=1).reshape(2 * nb, _B)

    csd = edges(content_edge_index)
    ssd = edges(social_edge_index)
    zrows = jnp.zeros((_RPS, _AW), jnp.float32)

    h = _gat_conv(content_x, csd, zrows, steps, n_valid,
                  Wc1, ac1s, ac1d, bc1, None)
    h = _gat_conv(h, csd, zrows, steps, n_valid,
                  Wc2, ac2s, ac2d, bc2, None)
    g = _gat_conv(social_x, ssd, zrows, steps, n_valid,
                  Ws1, as1s, as1d, bs1, social_x)
    g = _gat_conv(g, ssd, zrows, steps, n_valid,
                  Ws2, as2s, as2d, bs2, None)

    stats = _bn_stats(h, g)
    bnc = jnp.stack([bnc_g, bnc_b])
    bns = jnp.stack([bns_g, bns_b])

    pad_w = lambda w, rows, cols: jnp.zeros((rows, cols), jnp.float32).at[
        :w.shape[0], :w.shape[1]].set(w)
    pad_b = lambda b, cols: jnp.zeros((1, cols), jnp.float32).at[
        0, :b.shape[0]].set(b)

    out = _head(
        h, g, stats, bnc, bns,
        ca_w1, ca_b1.reshape(1, -1), pad_w(ca_w2, 64, 128),
        pad_b(ca_b2, 128), sa_w1, sa_b1.reshape(1, -1),
        pad_w(sa_w2, 64, 128), pad_b(sa_b2, 128),
        pad_w(fc1_w, 256, 128), pad_b(fc1_b, 128),
        pad_w(fc2_w, 128, 128), pad_b(fc2_b, 128),
        pad_w(fc3_w, 128, 128), pad_b(fc3_b, 128))
    return out[:, :40]


# speedup vs baseline: 1.1991x; 1.0944x over previous
import functools

import jax
import jax.numpy as jnp
import numpy as np
from jax import lax
from jax.experimental import pallas as pl
from jax.experimental.pallas import tpu as pltpu
from jax.experimental.pallas import tpu_sc as plsc

_N = 10000
_NC, _NS = 2, 16
_B = 80
_NPAD = 10112
_RPS = _NPAD // _NS
_AW = 144


def _mm_cat(x, w, expmat):
    n, k = x.shape
    f = w.shape[1]
    fe = expmat.shape[1]
    br = 1000

    def body(xr, wr, er, outr):
        xl = jnp.dot(xr[...], wr[...], preferred_element_type=jnp.float32)
        av = jnp.dot(xl, er[...], preferred_element_type=jnp.float32)
        outr[...] = jnp.concatenate([xl, av], axis=1)

    return pl.pallas_call(
        body,
        grid=(n // br,),
        in_specs=[
            pl.BlockSpec((br, k), lambda i: (i, 0)),
            pl.BlockSpec((k, f), lambda i: (0, 0)),
            pl.BlockSpec((f, fe), lambda i: (0, 0)),
        ],
        out_specs=pl.BlockSpec((br, f + fe), lambda i: (i, 0)),
        out_shape=jax.ShapeDtypeStruct((n, f + fe), jnp.float32),
    )(x, w, expmat)


def _epilogue(accs, exps, b, res):
    npass = len(accs)
    f = 128 * npass
    br = 1000
    nres = 1 if res is not None else 0

    def body(*refs):
        arefs = refs[:npass]
        erefs = refs[npass:2 * npass]
        bref = refs[2 * npass]
        rref = refs[2 * npass + 1] if nres else None
        outr = refs[-1]
        cols = []
        for p in range(npass):
            s = arefs[p][0] + arefs[p][1]
            den = jnp.dot(s[:, 128:136], erefs[p][...],
                          preferred_element_type=jnp.float32)
            cols.append(s[:, :128] / (den + 1e-16))
        x = cols[0] if npass == 1 else jnp.concatenate(cols, axis=1)
        x = x + bref[...]
        x = jnp.where(x > 0, x, jnp.exp(jnp.minimum(x, 0.0)) - 1.0)
        if nres:
            x = x + rref[...]
        outr[...] = x

    in_specs = (
        [pl.BlockSpec((2, br, _AW), lambda i: (0, i, 0)) for _ in range(npass)]
        + [pl.BlockSpec((8, 128), lambda i: (0, 0)) for _ in range(npass)]
        + [pl.BlockSpec((1, f), lambda i: (0, 0))]
        + ([pl.BlockSpec((br, f), lambda i: (i, 0))] if nres else [])
    )
    args = list(accs) + list(exps) + [b.reshape(1, f)] + ([res] if nres else [])
    return pl.pallas_call(
        body,
        grid=(_N // br,),
        in_specs=in_specs,
        out_specs=pl.BlockSpec((br, f), lambda i: (i, 0)),
        out_shape=jax.ShapeDtypeStruct((_N, f), jnp.float32),
    )(*args)


def _bn_stats(h, g):
    def body(hr, gr, outr):
        hv = hr[...]
        gv = gr[...]
        mu_h = jnp.mean(hv, axis=0, keepdims=True)
        mu_g = jnp.mean(gv, axis=0, keepdims=True)
        var_h = jnp.mean((hv - mu_h) ** 2, axis=0, keepdims=True)
        var_g = jnp.mean((gv - mu_g) ** 2, axis=0, keepdims=True)
        is_h = lax.rsqrt(var_h + 1e-5)
        is_g = lax.rsqrt(var_g + 1e-5)
        z = jnp.zeros_like(mu_h)
        outr[...] = jnp.concatenate(
            [mu_h, is_h, mu_g, is_g, z, z, z, z], axis=0)

    return pl.pallas_call(
        body,
        out_shape=jax.ShapeDtypeStruct((8, 128), jnp.float32),
    )(h, g)


def _head(h, g, stats, bnc, bns, caw1, cab1, caw2p, cab2, saw1, sab1, saw2p,
          sab2, fc1wp, fc1bp, fc2wp, fc2bp, fc3wp, fc3bp):
    br = 1000

    def body(hr, gr, str_, bncr, bnsr, cw1, cb1, cw2, cb2, sw1, sb1, sw2, sb2,
             f1w, f1b, f2w, f2b, f3w, f3b, outr):
        st = str_[...]
        hv = (hr[...] - st[0:1]) * st[1:2] * bncr[0:1] + bncr[1:2]
        gv = (gr[...] - st[2:3]) * st[3:4] * bnsr[0:1] + bnsr[1:2]
        cs = jnp.maximum(
            jnp.dot(hv, cw1[...], preferred_element_type=jnp.float32)
            + cb1[...], 0.0)
        cs = jnp.dot(cs, cw2[...], preferred_element_type=jnp.float32) + cb2[...]
        cs = jax.nn.sigmoid(cs[:, 0:1])
        ss = jnp.maximum(
            jnp.dot(gv, sw1[...], preferred_element_type=jnp.float32)
            + sb1[...], 0.0)
        ss = jnp.dot(ss, sw2[...], preferred_element_type=jnp.float32) + sb2[...]
        ss = jax.nn.sigmoid(ss[:, 0:1])
        mx = jnp.maximum(cs, ss)
        e0 = jnp.exp(cs - mx)
        e1 = jnp.exp(ss - mx)
        aw0 = e0 / (e0 + e1)
        aw1 = e1 / (e0 + e1)
        x = jnp.concatenate([hv * aw0, gv * aw1], axis=1)
        x = jnp.dot(x, f1w[...], preferred_element_type=jnp.float32) + f1b[...]
        x = jnp.where(x > 0, x, jnp.exp(jnp.minimum(x, 0.0)) - 1.0)
        x = jnp.dot(x, f2w[...], preferred_element_type=jnp.float32) + f2b[...]
        x = jnp.where(x > 0, x, jnp.exp(jnp.minimum(x, 0.0)) - 1.0)
        x = jnp.dot(x, f3w[...], preferred_element_type=jnp.float32) + f3b[...]
        col = lax.broadcasted_iota(jnp.int32, x.shape, 1)
        xm = jnp.where(col < 40, x, -1e30)
        m = jnp.max(xm, axis=1, keepdims=True)
        lse = jnp.log(jnp.sum(jnp.exp(xm - m), axis=1, keepdims=True))
        outr[...] = xm - m - lse

    full = lambda a: pl.BlockSpec(a.shape, lambda i: tuple(0 for _ in a.shape))
    weights = [caw1, cab1, caw2p, cab2, saw1, sab1, saw2p, sab2,
               fc1wp, fc1bp, fc2wp, fc2bp, fc3wp, fc3bp]
    return pl.pallas_call(
        body,
        grid=(_N // br,),
        in_specs=(
            [pl.BlockSpec((br, 128), lambda i: (i, 0)),
             pl.BlockSpec((br, 128), lambda i: (i, 0)),
             full(stats), full(bnc), full(bns)]
            + [full(w) for w in weights]),
        out_specs=pl.BlockSpec((br, 128), lambda i: (i, 0)),
        out_shape=jax.ShapeDtypeStruct((_N, 128), jnp.float32),
    )(h, g, stats, bnc, bns, *weights)


def _edge_pass(xcat, adst_t, sd_i, zrows, steps, head_of_vec):
    assert steps % 3 == 0
    mesh = plsc.VectorSubcoreMesh(core_axis_name="c", subcore_axis_name="s",
                                  num_cores=_NC, num_subcores=_NS)

    @functools.partial(
        pl.kernel,
        mesh=mesh,
        compiler_params=pltpu.CompilerParams(use_tc_tiling_on_sc=False),
        out_type=jax.ShapeDtypeStruct((_NC, _NPAD, _AW), jnp.float32),
        scratch_types=[
            pltpu.VMEM_SHARED((_NPAD, _AW), jnp.float32),
            pltpu.VMEM((3, 2, _B), jnp.int32),
            pltpu.VMEM((3, _B, _AW), jnp.float32),
            pltpu.VMEM((3, _B, 16), jnp.float32),
            pltpu.SemaphoreType.DMA,
            pltpu.SemaphoreType.DMA,
            pltpu.SemaphoreType.DMA,
            pltpu.SemaphoreType.DMA,
            pltpu.SemaphoreType.DMA,
            pltpu.SemaphoreType.DMA,
            pltpu.SemaphoreType.DMA,
            pltpu.SemaphoreType.DMA,
            pltpu.SemaphoreType.DMA,
        ],
    )
    def kfn(xcat_h, adst_h, sd_h, zr_h, out_h,
            acc, sdv, gbuf, dbuf, g0, g1, g2, d0, d1, d2, c0, c1, c2):
        cid = lax.axis_index("c")
        sid = lax.axis_index("s")
        pltpu.sync_copy(zr_h, acc.at[pl.ds(sid * _RPS, _RPS)])
        plsc.subcore_barrier()
        tile = cid * _NS + sid
        gsem = (g0, g1, g2)
        dsem = (d0, d1, d2)
        ssem = (c0, c1, c2)

        def fetch(i, b):
            g = tile * steps + i
            pltpu.sync_copy(sd_h.at[pl.ds(2 * g, 2)], sdv.at[b])
            pltpu.async_copy(xcat_h.at[sdv.at[b, 0]], gbuf.at[b], gsem[b])
            pltpu.async_copy(adst_h.at[sdv.at[b, 1]], dbuf.at[b], dsem[b])

        def drain_scatter(b):
            pltpu.make_async_copy(gbuf.at[b], acc.at[sdv.at[b, 1]],
                                  ssem[b]).wait()

        def step(i, b):
            bp = (b + 2) % 3
            pltpu.make_async_copy(xcat_h.at[sdv.at[b, 0]], gbuf.at[b],
                                  gsem[b]).wait()
            pltpu.make_async_copy(adst_h.at[sdv.at[b, 1]], dbuf.at[b],
                                  dsem[b]).wait()

            def edge(e, carry2):
                asv = gbuf[b, e, pl.ds(128, 16)]
                adv = dbuf[b, e, pl.ds(0, 16)]
                al = asv + adv
                al = jnp.maximum(al, al * 0.2)
                w = jnp.exp(al)
                for j in range(8):
                    wj = w[head_of_vec[j]]
                    gbuf[b, e, pl.ds(j * 16, 16)] = (
                        gbuf[b, e, pl.ds(j * 16, 16)] * wj)
                gbuf[b, e, pl.ds(128, 16)] = w
                return carry2

            lax.fori_loop(0, _B, edge, 0, unroll=4)

            @pl.when(i >= 1)
            def _():
                drain_scatter(bp)

            @pl.when(i + 2 < steps)
            def _():
                fetch(i + 2, bp)

            pltpu.async_copy(gbuf.at[b], acc.at[sdv.at[b, 1]], ssem[b],
                             add=True)

        fetch(0, 0)
        fetch(1, 1)

        def blk3(i3, carry):
            i = i3 * 3
            for b in range(3):
                step(i + b, b)
            return carry

        lax.fori_loop(0, steps // 3, blk3, 0)
        drain_scatter((steps - 1) % 3)
        plsc.subcore_barrier()
        pltpu.sync_copy(acc.at[pl.ds(sid * _RPS, _RPS)],
                        out_h.at[cid, pl.ds(sid * _RPS, _RPS)])

    return kfn(xcat, adst_t, sd_i, zrows)


def _place(a):
    h, c = a.shape
    out = jnp.zeros((h * c, h), jnp.float32)
    for i in range(h):
        out = out.at[i * c:(i + 1) * c, i].set(a[i])
    return out


def _head_exp(head_of_col):
    m = np.zeros((8, 128), np.float32)
    for c, h in enumerate(head_of_col):
        m[h, c] = 1.0
    return jnp.asarray(m)


def _gat_conv(x, sd_i, zrows, steps, n_valid, W, a_s, a_d, b, res):
    n = x.shape[0]
    f = W.shape[1]
    c = a_s.shape[1]
    expmat = jnp.concatenate([_place(a_s), _place(a_d)], axis=1)
    xlav = _mm_cat(x, W, expmat)
    prow = jnp.zeros((8, f + 16), jnp.float32).at[:, f:f + 8].set(-1e30)
    xlav = jnp.concatenate([xlav, prow], axis=0)
    zpad = jnp.zeros((n + 8, 8), jnp.float32)
    adst_t = jnp.concatenate([xlav[:, f + 8:f + 16], zpad], axis=1)
    npass = f // 128
    accs, exps = [], []
    for p in range(npass):
        xcat = jnp.concatenate(
            [xlav[:, p * 128:(p + 1) * 128], xlav[:, f:f + 8], zpad], axis=1)
        head_of_vec = tuple((p * 128 + 16 * j) // c for j in range(8))
        acc = _edge_pass(xcat, adst_t, sd_i, zrows, steps, head_of_vec)
        accs.append(acc[:, :n, :])
        exps.append(_head_exp([(p * 128 + cc) // c for cc in range(128)]))
    return _epilogue(accs, exps, b, res)


def kernel(content_x, social_x, content_edge_index, social_edge_index,
           Wc1, ac1s, ac1d, bc1, Wc2, ac2s, ac2d, bc2,
           Ws1, as1s, as1d, bs1, Ws2, as2s, as2d, bs2,
           ca_w1, ca_b1, ca_w2, ca_b2, sa_w1, sa_b1, sa_w2, sa_b2,
           bnc_g, bnc_b, bns_g, bns_b,
           fc1_w, fc1_b, fc2_w, fc2_b, fc3_w, fc3_b):
    n = content_x.shape[0]
    e = content_edge_index.shape[1]
    n_valid = e + n
    steps = -(-n_valid // (_NC * _NS * _B))
    steps = -(-steps // 3) * 3
    ep = _NC * _NS * steps * _B
    loop = jnp.arange(n, dtype=content_edge_index.dtype)
    pad = jnp.full((ep - n_valid,), n, content_edge_index.dtype)

    def edges(ei):
        s = jnp.concatenate([ei[0], loop, pad])
        d = jnp.concatenate([ei[1], loop, pad])
        nb = ep // _B
        return jnp.stack([s.reshape(nb, _B), d.reshape(nb, _B)],
                         axis=1).reshape(2 * nb, _B)

    csd = edges(content_edge_index)
    ssd = edges(social_edge_index)
    zrows = jnp.zeros((_RPS, _AW), jnp.float32)

    h = _gat_conv(content_x, csd, zrows, steps, n_valid,
                  Wc1, ac1s, ac1d, bc1, None)
    h = _gat_conv(h, csd, zrows, steps, n_valid,
                  Wc2, ac2s, ac2d, bc2, None)
    g = _gat_conv(social_x, ssd, zrows, steps, n_valid,
                  Ws1, as1s, as1d, bs1, social_x)
    g = _gat_conv(g, ssd, zrows, steps, n_valid,
                  Ws2, as2s, as2d, bs2, None)

    stats = _bn_stats(h, g)
    bnc = jnp.stack([bnc_g, bnc_b])
    bns = jnp.stack([bns_g, bns_b])

    pad_w = lambda w, rows, cols: jnp.zeros((rows, cols), jnp.float32).at[
        :w.shape[0], :w.shape[1]].set(w)
    pad_b = lambda b, cols: jnp.zeros((1, cols), jnp.float32).at[
        0, :b.shape[0]].set(b)

    out = _head(
        h, g, stats, bnc, bns,
        ca_w1, ca_b1.reshape(1, -1), pad_w(ca_w2, 64, 128),
        pad_b(ca_b2, 128), sa_w1, sa_b1.reshape(1, -1),
        pad_w(sa_w2, 64, 128), pad_b(sa_b2, 128),
        pad_w(fc1_w, 256, 128), pad_b(fc1_b, 128),
        pad_w(fc2_w, 128, 128), pad_b(fc2_b, 128),
        pad_w(fc3_w, 128, 128), pad_b(fc3_b, 128))
    return out[:, :40]


# speedup vs baseline: 1.2939x; 1.0791x over previous
import functools

import jax
import jax.numpy as jnp
import numpy as np
from jax import lax
from jax.experimental import pallas as pl
from jax.experimental.pallas import tpu as pltpu
from jax.experimental.pallas import tpu_sc as plsc

_N = 10000
_NC, _NS = 2, 16
_B = 80
_NPAD = 10112
_RPS = _NPAD // _NS
_AW = 144


def _mm_tables(x, w, expmat):
    n, k = x.shape
    f = w.shape[1]
    npass = f // 128
    br = 1000

    def body(xr, wr, er, *outs):
        xl = jnp.dot(xr[...], wr[...], preferred_element_type=jnp.float32)
        av = jnp.dot(xl, er[...], preferred_element_type=jnp.float32)
        z8 = jnp.zeros((br, 8), jnp.float32)
        for p in range(npass):
            outs[p][...] = jnp.concatenate(
                [xl[:, p * 128:(p + 1) * 128], av[:, :8], z8], axis=1)
        outs[npass][...] = jnp.concatenate([av[:, 8:16], z8], axis=1)

    outs = pl.pallas_call(
        body,
        grid=(n // br,),
        in_specs=[
            pl.BlockSpec((br, k), lambda i: (i, 0)),
            pl.BlockSpec((k, f), lambda i: (0, 0)),
            pl.BlockSpec((f, 16), lambda i: (0, 0)),
        ],
        out_specs=[pl.BlockSpec((br, _AW), lambda i: (i, 0))
                   for _ in range(npass)]
        + [pl.BlockSpec((br, 16), lambda i: (i, 0))],
        out_shape=[jax.ShapeDtypeStruct((n, _AW), jnp.float32)
                   for _ in range(npass)]
        + [jax.ShapeDtypeStruct((n, 16), jnp.float32)],
    )(x, w, expmat)
    return outs[:npass], outs[npass]


def _epilogue(accs, exps, b, res):
    npass = len(accs)
    f = 128 * npass
    br = 1000
    nres = 1 if res is not None else 0

    def body(*refs):
        arefs = refs[:npass]
        erefs = refs[npass:2 * npass]
        bref = refs[2 * npass]
        rref = refs[2 * npass + 1] if nres else None
        outr = refs[-1]
        cols = []
        for p in range(npass):
            s = arefs[p][0] + arefs[p][1]
            den = jnp.dot(s[:, 128:136], erefs[p][...],
                          preferred_element_type=jnp.float32)
            cols.append(s[:, :128] / (den + 1e-16))
        x = cols[0] if npass == 1 else jnp.concatenate(cols, axis=1)
        x = x + bref[...]
        x = jnp.where(x > 0, x, jnp.exp(jnp.minimum(x, 0.0)) - 1.0)
        if nres:
            x = x + rref[...]
        outr[...] = x

    in_specs = (
        [pl.BlockSpec((2, br, _AW), lambda i: (0, i, 0)) for _ in range(npass)]
        + [pl.BlockSpec((8, 128), lambda i: (0, 0)) for _ in range(npass)]
        + [pl.BlockSpec((1, f), lambda i: (0, 0))]
        + ([pl.BlockSpec((br, f), lambda i: (i, 0))] if nres else [])
    )
    args = list(accs) + list(exps) + [b.reshape(1, f)] + ([res] if nres else [])
    return pl.pallas_call(
        body,
        grid=(_N // br,),
        in_specs=in_specs,
        out_specs=pl.BlockSpec((br, f), lambda i: (i, 0)),
        out_shape=jax.ShapeDtypeStruct((_N, f), jnp.float32),
    )(*args)


def _bn_stats(h, g):
    def body(hr, gr, outr):
        hv = hr[...]
        gv = gr[...]
        mu_h = jnp.mean(hv, axis=0, keepdims=True)
        mu_g = jnp.mean(gv, axis=0, keepdims=True)
        var_h = jnp.mean((hv - mu_h) ** 2, axis=0, keepdims=True)
        var_g = jnp.mean((gv - mu_g) ** 2, axis=0, keepdims=True)
        is_h = lax.rsqrt(var_h + 1e-5)
        is_g = lax.rsqrt(var_g + 1e-5)
        z = jnp.zeros_like(mu_h)
        outr[...] = jnp.concatenate(
            [mu_h, is_h, mu_g, is_g, z, z, z, z], axis=0)

    return pl.pallas_call(
        body,
        out_shape=jax.ShapeDtypeStruct((8, 128), jnp.float32),
    )(h, g)


def _head(h, g, stats, bnc, bns, caw1, cab1, caw2p, cab2, saw1, sab1, saw2p,
          sab2, fc1wp, fc1bp, fc2wp, fc2bp, fc3wp, fc3bp):
    br = 1000

    def body(hr, gr, str_, bncr, bnsr, cw1, cb1, cw2, cb2, sw1, sb1, sw2, sb2,
             f1w, f1b, f2w, f2b, f3w, f3b, outr):
        st = str_[...]
        hv = (hr[...] - st[0:1]) * st[1:2] * bncr[0:1] + bncr[1:2]
        gv = (gr[...] - st[2:3]) * st[3:4] * bnsr[0:1] + bnsr[1:2]
        cs = jnp.maximum(
            jnp.dot(hv, cw1[...], preferred_element_type=jnp.float32)
            + cb1[...], 0.0)
        cs = jnp.dot(cs, cw2[...], preferred_element_type=jnp.float32) + cb2[...]
        cs = jax.nn.sigmoid(cs[:, 0:1])
        ss = jnp.maximum(
            jnp.dot(gv, sw1[...], preferred_element_type=jnp.float32)
            + sb1[...], 0.0)
        ss = jnp.dot(ss, sw2[...], preferred_element_type=jnp.float32) + sb2[...]
        ss = jax.nn.sigmoid(ss[:, 0:1])
        mx = jnp.maximum(cs, ss)
        e0 = jnp.exp(cs - mx)
        e1 = jnp.exp(ss - mx)
        aw0 = e0 / (e0 + e1)
        aw1 = e1 / (e0 + e1)
        x = jnp.concatenate([hv * aw0, gv * aw1], axis=1)
        x = jnp.dot(x, f1w[...], preferred_element_type=jnp.float32) + f1b[...]
        x = jnp.where(x > 0, x, jnp.exp(jnp.minimum(x, 0.0)) - 1.0)
        x = jnp.dot(x, f2w[...], preferred_element_type=jnp.float32) + f2b[...]
        x = jnp.where(x > 0, x, jnp.exp(jnp.minimum(x, 0.0)) - 1.0)
        x = jnp.dot(x, f3w[...], preferred_element_type=jnp.float32) + f3b[...]
        col = lax.broadcasted_iota(jnp.int32, x.shape, 1)
        xm = jnp.where(col < 40, x, -1e30)
        m = jnp.max(xm, axis=1, keepdims=True)
        lse = jnp.log(jnp.sum(jnp.exp(xm - m), axis=1, keepdims=True))
        outr[...] = xm - m - lse

    full = lambda a: pl.BlockSpec(a.shape, lambda i: tuple(0 for _ in a.shape))
    weights = [caw1, cab1, caw2p, cab2, saw1, sab1, saw2p, sab2,
               fc1wp, fc1bp, fc2wp, fc2bp, fc3wp, fc3bp]
    return pl.pallas_call(
        body,
        grid=(_N // br,),
        in_specs=(
            [pl.BlockSpec((br, 128), lambda i: (i, 0)),
             pl.BlockSpec((br, 128), lambda i: (i, 0)),
             full(stats), full(bnc), full(bns)]
            + [full(w) for w in weights]),
        out_specs=pl.BlockSpec((br, 128), lambda i: (i, 0)),
        out_shape=jax.ShapeDtypeStruct((_N, 128), jnp.float32),
    )(h, g, stats, bnc, bns, *weights)


def _edge_pass(xcat, adst_t, sd_i, zrows, steps, head_of_vec):
    assert steps % 3 == 0
    mesh = plsc.VectorSubcoreMesh(core_axis_name="c", subcore_axis_name="s",
                                  num_cores=_NC, num_subcores=_NS)

    @functools.partial(
        pl.kernel,
        mesh=mesh,
        compiler_params=pltpu.CompilerParams(use_tc_tiling_on_sc=False),
        out_type=jax.ShapeDtypeStruct((_NC, _NPAD, _AW), jnp.float32),
        scratch_types=[
            pltpu.VMEM_SHARED((_NPAD, _AW), jnp.float32),
            pltpu.VMEM((3, 2, _B), jnp.int32),
            pltpu.VMEM((3, _B, _AW), jnp.float32),
            pltpu.VMEM((3, _B, 16), jnp.float32),
            pltpu.SemaphoreType.DMA,
            pltpu.SemaphoreType.DMA,
            pltpu.SemaphoreType.DMA,
            pltpu.SemaphoreType.DMA,
            pltpu.SemaphoreType.DMA,
            pltpu.SemaphoreType.DMA,
            pltpu.SemaphoreType.DMA,
            pltpu.SemaphoreType.DMA,
            pltpu.SemaphoreType.DMA,
        ],
    )
    def kfn(xcat_h, adst_h, sd_h, zr_h, out_h,
            acc, sdv, gbuf, dbuf, g0, g1, g2, d0, d1, d2, c0, c1, c2):
        cid = lax.axis_index("c")
        sid = lax.axis_index("s")
        pltpu.sync_copy(zr_h, acc.at[pl.ds(sid * _RPS, _RPS)])
        plsc.subcore_barrier()
        tile = cid * _NS + sid
        gsem = (g0, g1, g2)
        dsem = (d0, d1, d2)
        ssem = (c0, c1, c2)

        def fetch(i, b):
            g = tile * steps + i
            pltpu.sync_copy(sd_h.at[pl.ds(2 * g, 2)], sdv.at[b])
            pltpu.async_copy(xcat_h.at[sdv.at[b, 0]], gbuf.at[b], gsem[b])
            pltpu.async_copy(adst_h.at[sdv.at[b, 1]], dbuf.at[b], dsem[b])

        def drain_scatter(b):
            pltpu.make_async_copy(gbuf.at[b], acc.at[sdv.at[b, 1]],
                                  ssem[b]).wait()

        def step(i, b):
            bp = (b + 2) % 3
            pltpu.make_async_copy(xcat_h.at[sdv.at[b, 0]], gbuf.at[b],
                                  gsem[b]).wait()
            pltpu.make_async_copy(adst_h.at[sdv.at[b, 1]], dbuf.at[b],
                                  dsem[b]).wait()

            def edge(e, carry2):
                asv = gbuf[b, e, pl.ds(128, 16)]
                adv = dbuf[b, e, pl.ds(0, 16)]
                al = asv + adv
                al = jnp.maximum(al, al * 0.2)
                w = jnp.exp(al)
                for j in range(8):
                    wj = w[head_of_vec[j]]
                    gbuf[b, e, pl.ds(j * 16, 16)] = (
                        gbuf[b, e, pl.ds(j * 16, 16)] * wj)
                gbuf[b, e, pl.ds(128, 16)] = w
                return carry2

            lax.fori_loop(0, _B, edge, 0, unroll=4)

            @pl.when(i >= 1)
            def _():
                drain_scatter(bp)

            @pl.when(i + 2 < steps)
            def _():
                fetch(i + 2, bp)

            pltpu.async_copy(gbuf.at[b], acc.at[sdv.at[b, 1]], ssem[b],
                             add=True)

        fetch(0, 0)
        fetch(1, 1)

        def blk3(i3, carry):
            i = i3 * 3
            for b in range(3):
                step(i + b, b)
            return carry

        lax.fori_loop(0, steps // 3, blk3, 0)
        drain_scatter((steps - 1) % 3)
        plsc.subcore_barrier()
        pltpu.sync_copy(acc.at[pl.ds(sid * _RPS, _RPS)],
                        out_h.at[cid, pl.ds(sid * _RPS, _RPS)])

    return kfn(xcat, adst_t, sd_i, zrows)


def _place(a):
    h, c = a.shape
    out = jnp.zeros((h * c, h), jnp.float32)
    for i in range(h):
        out = out.at[i * c:(i + 1) * c, i].set(a[i])
    return out


def _head_exp(head_of_col):
    m = np.zeros((8, 128), np.float32)
    for c, h in enumerate(head_of_col):
        m[h, c] = 1.0
    return jnp.asarray(m)


def _gat_conv(x, sd_i, zrows, steps, n_valid, W, a_s, a_d, b, res):
    n = x.shape[0]
    f = W.shape[1]
    c = a_s.shape[1]
    expmat = jnp.concatenate([_place(a_s), _place(a_d)], axis=1)
    xcats, adst_t = _mm_tables(x, W, expmat)
    npass = f // 128
    accs, exps = [], []
    for p in range(npass):
        head_of_vec = tuple((p * 128 + 16 * j) // c for j in range(8))
        acc = _edge_pass(xcats[p], adst_t, sd_i, zrows, steps, head_of_vec)
        accs.append(acc[:, :n, :])
        exps.append(_head_exp([(p * 128 + cc) // c for cc in range(128)]))
    return _epilogue(accs, exps, b, res)


def kernel(content_x, social_x, content_edge_index, social_edge_index,
           Wc1, ac1s, ac1d, bc1, Wc2, ac2s, ac2d, bc2,
           Ws1, as1s, as1d, bs1, Ws2, as2s, as2d, bs2,
           ca_w1, ca_b1, ca_w2, ca_b2, sa_w1, sa_b1, sa_w2, sa_b2,
           bnc_g, bnc_b, bns_g, bns_b,
           fc1_w, fc1_b, fc2_w, fc2_b, fc3_w, fc3_b):
    n = content_x.shape[0]
    e = content_edge_index.shape[1]
    n_valid = e + n
    steps = -(-n_valid // (_NC * _NS * _B))
    steps = -(-steps // 3) * 3
    ep = _NC * _NS * steps * _B
    loop = jnp.arange(n, dtype=content_edge_index.dtype)
    spad = jnp.zeros((ep - n_valid,), content_edge_index.dtype)
    dpad = jnp.full((ep - n_valid,), n, content_edge_index.dtype)

    def edges(ei):
        s = jnp.concatenate([ei[0], loop, spad])
        d = jnp.concatenate([ei[1], loop, dpad])
        nb = ep // _B
        return jnp.stack([s.reshape(nb, _B), d.reshape(nb, _B)],
                         axis=1).reshape(2 * nb, _B)

    csd = edges(content_edge_index)
    ssd = edges(social_edge_index)
    zrows = jnp.zeros((_RPS, _AW), jnp.float32)

    h = _gat_conv(content_x, csd, zrows, steps, n_valid,
                  Wc1, ac1s, ac1d, bc1, None)
    h = _gat_conv(h, csd, zrows, steps, n_valid,
                  Wc2, ac2s, ac2d, bc2, None)
    g = _gat_conv(social_x, ssd, zrows, steps, n_valid,
                  Ws1, as1s, as1d, bs1, social_x)
    g = _gat_conv(g, ssd, zrows, steps, n_valid,
                  Ws2, as2s, as2d, bs2, None)

    stats = _bn_stats(h, g)
    bnc = jnp.stack([bnc_g, bnc_b])
    bns = jnp.stack([bns_g, bns_b])

    pad_w = lambda w, rows, cols: jnp.zeros((rows, cols), jnp.float32).at[
        :w.shape[0], :w.shape[1]].set(w)
    pad_b = lambda b, cols: jnp.zeros((1, cols), jnp.float32).at[
        0, :b.shape[0]].set(b)

    out = _head(
        h, g, stats, bnc, bns,
        ca_w1, ca_b1.reshape(1, -1), pad_w(ca_w2, 64, 128),
        pad_b(ca_b2, 128), sa_w1, sa_b1.reshape(1, -1),
        pad_w(sa_w2, 64, 128), pad_b(sa_b2, 128),
        pad_w(fc1_w, 256, 128), pad_b(fc1_b, 128),
        pad_w(fc2_w, 128, 128), pad_b(fc2_b, 128),
        pad_w(fc3_w, 128, 128), pad_b(fc3_b, 128))
    return out[:, :40]


# speedup vs baseline: 1.3005x; 1.0051x over previous
import functools

import jax
import jax.numpy as jnp
import numpy as np
from jax import lax
from jax.experimental import pallas as pl
from jax.experimental.pallas import tpu as pltpu
from jax.experimental.pallas import tpu_sc as plsc

_N = 10000
_NC, _NS = 2, 16
_B = 80
_NPAD = 10112
_RPS = _NPAD // _NS
_AW = 144


def _mm_tables(x, w, expmat):
    n, k = x.shape
    f = w.shape[1]
    npass = f // 128
    br = 1000

    def body(xr, wr, er, *outs):
        xl = jnp.dot(xr[...], wr[...], preferred_element_type=jnp.float32)
        av = jnp.dot(xl, er[...], preferred_element_type=jnp.float32)
        z8 = jnp.zeros((br, 8), jnp.float32)
        for p in range(npass):
            outs[p][...] = jnp.concatenate(
                [xl[:, p * 128:(p + 1) * 128], av[:, :8], z8], axis=1)
        outs[npass][...] = jnp.concatenate([av[:, 8:16], z8], axis=1)

    outs = pl.pallas_call(
        body,
        grid=(n // br,),
        in_specs=[
            pl.BlockSpec((br, k), lambda i: (i, 0)),
            pl.BlockSpec((k, f), lambda i: (0, 0)),
            pl.BlockSpec((f, 16), lambda i: (0, 0)),
        ],
        out_specs=[pl.BlockSpec((br, _AW), lambda i: (i, 0))
                   for _ in range(npass)]
        + [pl.BlockSpec((br, 16), lambda i: (i, 0))],
        out_shape=[jax.ShapeDtypeStruct((n, _AW), jnp.float32)
                   for _ in range(npass)]
        + [jax.ShapeDtypeStruct((n, 16), jnp.float32)],
    )(x, w, expmat)
    return outs[:npass], outs[npass]


def _epilogue(accs, exps, b, res):
    npass = len(accs)
    f = 128 * npass
    br = 1000
    nres = 1 if res is not None else 0

    def body(*refs):
        arefs = refs[:npass]
        erefs = refs[npass:2 * npass]
        bref = refs[2 * npass]
        rref = refs[2 * npass + 1] if nres else None
        outr = refs[-1]
        cols = []
        for p in range(npass):
            s = arefs[p][0] + arefs[p][1]
            den = jnp.dot(s[:, 128:136], erefs[p][...],
                          preferred_element_type=jnp.float32)
            cols.append(s[:, :128] / (den + 1e-16))
        x = cols[0] if npass == 1 else jnp.concatenate(cols, axis=1)
        x = x + bref[...]
        x = jnp.where(x > 0, x, jnp.exp(jnp.minimum(x, 0.0)) - 1.0)
        if nres:
            x = x + rref[...]
        outr[...] = x

    in_specs = (
        [pl.BlockSpec((2, br, _AW), lambda i: (0, i, 0)) for _ in range(npass)]
        + [pl.BlockSpec((8, 128), lambda i: (0, 0)) for _ in range(npass)]
        + [pl.BlockSpec((1, f), lambda i: (0, 0))]
        + ([pl.BlockSpec((br, f), lambda i: (i, 0))] if nres else [])
    )
    args = list(accs) + list(exps) + [b.reshape(1, f)] + ([res] if nres else [])
    return pl.pallas_call(
        body,
        grid=(_N // br,),
        in_specs=in_specs,
        out_specs=pl.BlockSpec((br, f), lambda i: (i, 0)),
        out_shape=jax.ShapeDtypeStruct((_N, f), jnp.float32),
    )(*args)


def _bn_stats(h, g):
    def body(hr, gr, outr):
        hv = hr[...]
        gv = gr[...]
        mu_h = jnp.mean(hv, axis=0, keepdims=True)
        mu_g = jnp.mean(gv, axis=0, keepdims=True)
        var_h = jnp.mean((hv - mu_h) ** 2, axis=0, keepdims=True)
        var_g = jnp.mean((gv - mu_g) ** 2, axis=0, keepdims=True)
        is_h = lax.rsqrt(var_h + 1e-5)
        is_g = lax.rsqrt(var_g + 1e-5)
        z = jnp.zeros_like(mu_h)
        outr[...] = jnp.concatenate(
            [mu_h, is_h, mu_g, is_g, z, z, z, z], axis=0)

    return pl.pallas_call(
        body,
        out_shape=jax.ShapeDtypeStruct((8, 128), jnp.float32),
    )(h, g)


def _head(h, g, stats, bnc, bns, caw1, cab1, caw2p, cab2, saw1, sab1, saw2p,
          sab2, fc1wp, fc1bp, fc2wp, fc2bp, fc3wp, fc3bp):
    br = 1000

    def body(hr, gr, str_, bncr, bnsr, cw1, cb1, cw2, cb2, sw1, sb1, sw2, sb2,
             f1w, f1b, f2w, f2b, f3w, f3b, outr):
        st = str_[...]
        hv = (hr[...] - st[0:1]) * st[1:2] * bncr[0:1] + bncr[1:2]
        gv = (gr[...] - st[2:3]) * st[3:4] * bnsr[0:1] + bnsr[1:2]
        cs = jnp.maximum(
            jnp.dot(hv, cw1[...], preferred_element_type=jnp.float32)
            + cb1[...], 0.0)
        cs = jnp.dot(cs, cw2[...], preferred_element_type=jnp.float32) + cb2[...]
        cs = jax.nn.sigmoid(cs[:, 0:1])
        ss = jnp.maximum(
            jnp.dot(gv, sw1[...], preferred_element_type=jnp.float32)
            + sb1[...], 0.0)
        ss = jnp.dot(ss, sw2[...], preferred_element_type=jnp.float32) + sb2[...]
        ss = jax.nn.sigmoid(ss[:, 0:1])
        mx = jnp.maximum(cs, ss)
        e0 = jnp.exp(cs - mx)
        e1 = jnp.exp(ss - mx)
        aw0 = e0 / (e0 + e1)
        aw1 = e1 / (e0 + e1)
        x = jnp.concatenate([hv * aw0, gv * aw1], axis=1)
        x = jnp.dot(x, f1w[...], preferred_element_type=jnp.float32) + f1b[...]
        x = jnp.where(x > 0, x, jnp.exp(jnp.minimum(x, 0.0)) - 1.0)
        x = jnp.dot(x, f2w[...], preferred_element_type=jnp.float32) + f2b[...]
        x = jnp.where(x > 0, x, jnp.exp(jnp.minimum(x, 0.0)) - 1.0)
        x = jnp.dot(x, f3w[...], preferred_element_type=jnp.float32) + f3b[...]
        col = lax.broadcasted_iota(jnp.int32, x.shape, 1)
        xm = jnp.where(col < 40, x, -1e30)
        m = jnp.max(xm, axis=1, keepdims=True)
        lse = jnp.log(jnp.sum(jnp.exp(xm - m), axis=1, keepdims=True))
        outr[...] = xm - m - lse

    full = lambda a: pl.BlockSpec(a.shape, lambda i: tuple(0 for _ in a.shape))
    weights = [caw1, cab1, caw2p, cab2, saw1, sab1, saw2p, sab2,
               fc1wp, fc1bp, fc2wp, fc2bp, fc3wp, fc3bp]
    return pl.pallas_call(
        body,
        grid=(_N // br,),
        in_specs=(
            [pl.BlockSpec((br, 128), lambda i: (i, 0)),
             pl.BlockSpec((br, 128), lambda i: (i, 0)),
             full(stats), full(bnc), full(bns)]
            + [full(w) for w in weights]),
        out_specs=pl.BlockSpec((br, 128), lambda i: (i, 0)),
        out_shape=jax.ShapeDtypeStruct((_N, 128), jnp.float32),
    )(h, g, stats, bnc, bns, *weights)


def _edge_pass(xcat, adst_t, sd_i, zrows, steps, head_of_vec):
    assert steps % 3 == 0
    mesh = plsc.VectorSubcoreMesh(core_axis_name="c", subcore_axis_name="s",
                                  num_cores=_NC, num_subcores=_NS)

    @functools.partial(
        pl.kernel,
        mesh=mesh,
        compiler_params=pltpu.CompilerParams(use_tc_tiling_on_sc=False),
        out_type=jax.ShapeDtypeStruct((_NC, _NPAD, _AW), jnp.float32),
        scratch_types=[
            pltpu.VMEM_SHARED((_NPAD, _AW), jnp.float32),
            pltpu.VMEM((3, 2, _B), jnp.int32),
            pltpu.VMEM((3, _B, _AW), jnp.float32),
            pltpu.VMEM((3, _B, 16), jnp.float32),
            pltpu.SemaphoreType.DMA,
            pltpu.SemaphoreType.DMA,
            pltpu.SemaphoreType.DMA,
            pltpu.SemaphoreType.DMA,
            pltpu.SemaphoreType.DMA,
            pltpu.SemaphoreType.DMA,
            pltpu.SemaphoreType.DMA,
            pltpu.SemaphoreType.DMA,
            pltpu.SemaphoreType.DMA,
        ],
    )
    def kfn(xcat_h, adst_h, sd_h, zr_h, out_h,
            acc, sdv, gbuf, dbuf, g0, g1, g2, d0, d1, d2, c0, c1, c2):
        cid = lax.axis_index("c")
        sid = lax.axis_index("s")
        pltpu.sync_copy(zr_h, acc.at[pl.ds(sid * _RPS, _RPS)])
        plsc.subcore_barrier()
        tile = cid * _NS + sid
        gsem = (g0, g1, g2)
        dsem = (d0, d1, d2)
        ssem = (c0, c1, c2)

        def fetch(i, b):
            g = tile * steps + i
            pltpu.sync_copy(sd_h.at[pl.ds(2 * g, 2)], sdv.at[b])
            pltpu.async_copy(xcat_h.at[sdv.at[b, 0]], gbuf.at[b], gsem[b])
            pltpu.async_copy(adst_h.at[sdv.at[b, 1]], dbuf.at[b], dsem[b])

        def drain_scatter(b):
            pltpu.make_async_copy(gbuf.at[b], acc.at[sdv.at[b, 1]],
                                  ssem[b]).wait()

        def step(i, b):
            bp = (b + 2) % 3
            pltpu.make_async_copy(xcat_h.at[sdv.at[b, 0]], gbuf.at[b],
                                  gsem[b]).wait()
            pltpu.make_async_copy(adst_h.at[sdv.at[b, 1]], dbuf.at[b],
                                  dsem[b]).wait()

            def edge(e, carry2):
                asv = gbuf[b, e, pl.ds(128, 16)]
                adv = dbuf[b, e, pl.ds(0, 16)]
                al = asv + adv
                al = jnp.maximum(al, al * 0.2)
                w = jnp.exp(al)
                for j in range(8):
                    wj = w[head_of_vec[j]]
                    gbuf[b, e, pl.ds(j * 16, 16)] = (
                        gbuf[b, e, pl.ds(j * 16, 16)] * wj)
                gbuf[b, e, pl.ds(128, 16)] = w
                return carry2

            lax.fori_loop(0, _B, edge, 0, unroll=2)

            @pl.when(i >= 1)
            def _():
                drain_scatter(bp)

            @pl.when(i + 2 < steps)
            def _():
                fetch(i + 2, bp)

            pltpu.async_copy(gbuf.at[b], acc.at[sdv.at[b, 1]], ssem[b],
                             add=True)

        fetch(0, 0)
        fetch(1, 1)

        def blk3(i3, carry):
            i = i3 * 3
            for b in range(3):
                step(i + b, b)
            return carry

        lax.fori_loop(0, steps // 3, blk3, 0)
        drain_scatter((steps - 1) % 3)
        plsc.subcore_barrier()
        pltpu.sync_copy(acc.at[pl.ds(sid * _RPS, _RPS)],
                        out_h.at[cid, pl.ds(sid * _RPS, _RPS)])

    return kfn(xcat, adst_t, sd_i, zrows)


def _place(a):
    h, c = a.shape
    out = jnp.zeros((h * c, h), jnp.float32)
    for i in range(h):
        out = out.at[i * c:(i + 1) * c, i].set(a[i])
    return out


def _head_exp(head_of_col):
    m = np.zeros((8, 128), np.float32)
    for c, h in enumerate(head_of_col):
        m[h, c] = 1.0
    return jnp.asarray(m)


def _gat_conv(x, sd_i, zrows, steps, n_valid, W, a_s, a_d, b, res):
    n = x.shape[0]
    f = W.shape[1]
    c = a_s.shape[1]
    expmat = jnp.concatenate([_place(a_s), _place(a_d)], axis=1)
    xcats, adst_t = _mm_tables(x, W, expmat)
    npass = f // 128
    accs, exps = [], []
    for p in range(npass):
        head_of_vec = tuple((p * 128 + 16 * j) // c for j in range(8))
        acc = _edge_pass(xcats[p], adst_t, sd_i, zrows, steps, head_of_vec)
        accs.append(acc[:, :n, :])
        exps.append(_head_exp([(p * 128 + cc) // c for cc in range(128)]))
    return _epilogue(accs, exps, b, res)


def kernel(content_x, social_x, content_edge_index, social_edge_index,
           Wc1, ac1s, ac1d, bc1, Wc2, ac2s, ac2d, bc2,
           Ws1, as1s, as1d, bs1, Ws2, as2s, as2d, bs2,
           ca_w1, ca_b1, ca_w2, ca_b2, sa_w1, sa_b1, sa_w2, sa_b2,
           bnc_g, bnc_b, bns_g, bns_b,
           fc1_w, fc1_b, fc2_w, fc2_b, fc3_w, fc3_b):
    n = content_x.shape[0]
    e = content_edge_index.shape[1]
    n_valid = e + n
    steps = -(-n_valid // (_NC * _NS * _B))
    steps = -(-steps // 3) * 3
    ep = _NC * _NS * steps * _B
    loop = jnp.arange(n, dtype=content_edge_index.dtype)
    spad = jnp.zeros((ep - n_valid,), content_edge_index.dtype)
    dpad = jnp.full((ep - n_valid,), n, content_edge_index.dtype)

    def edges(ei):
        s = jnp.concatenate([ei[0], loop, spad])
        d = jnp.concatenate([ei[1], loop, dpad])
        nb = ep // _B
        return jnp.stack([s.reshape(nb, _B), d.reshape(nb, _B)],
                         axis=1).reshape(2 * nb, _B)

    csd = edges(content_edge_index)
    ssd = edges(social_edge_index)
    zrows = jnp.zeros((_RPS, _AW), jnp.float32)

    h = _gat_conv(content_x, csd, zrows, steps, n_valid,
                  Wc1, ac1s, ac1d, bc1, None)
    h = _gat_conv(h, csd, zrows, steps, n_valid,
                  Wc2, ac2s, ac2d, bc2, None)
    g = _gat_conv(social_x, ssd, zrows, steps, n_valid,
                  Ws1, as1s, as1d, bs1, social_x)
    g = _gat_conv(g, ssd, zrows, steps, n_valid,
                  Ws2, as2s, as2d, bs2, None)

    stats = _bn_stats(h, g)
    bnc = jnp.stack([bnc_g, bnc_b])
    bns = jnp.stack([bns_g, bns_b])

    pad_w = lambda w, rows, cols: jnp.zeros((rows, cols), jnp.float32).at[
        :w.shape[0], :w.shape[1]].set(w)
    pad_b = lambda b, cols: jnp.zeros((1, cols), jnp.float32).at[
        0, :b.shape[0]].set(b)

    out = _head(
        h, g, stats, bnc, bns,
        ca_w1, ca_b1.reshape(1, -1), pad_w(ca_w2, 64, 128),
        pad_b(ca_b2, 128), sa_w1, sa_b1.reshape(1, -1),
        pad_w(sa_w2, 64, 128), pad_b(sa_b2, 128),
        pad_w(fc1_w, 256, 128), pad_b(fc1_b, 128),
        pad_w(fc2_w, 128, 128), pad_b(fc2_b, 128),
        pad_w(fc3_w, 128, 128), pad_b(fc3_b, 128))
    return out[:, :40]
